# baseline (speedup 1.0000x reference)
import jax
import jax.numpy as jnp
from jax.experimental import pallas as pl
from jax.experimental.pallas import tpu as pltpu

_BK = 256
_DN = (((1,), (1,)), ((), ()))


def _mlp_kernel(x_ref, w1_ref, b1_ref, w2_ref, b2_ref, w3_ref, b3_ref,
                o_ref, h1_ref, h2_ref):
    s = pl.program_id(0)
    k = pl.num_programs(0) // 2

    @pl.when(s < k)
    def _phase1():
        h = jax.lax.dot_general(x_ref[...], w1_ref[...], _DN,
                                preferred_element_type=jnp.float32)
        h1_ref[:, pl.ds(s * _BK, _BK)] = jnp.maximum(
            h + b1_ref[:, pl.ds(s * _BK, _BK)], 0.0)

    @pl.when(s >= k)
    def _phase2():
        j = s - k
        h = jax.lax.dot_general(h1_ref[...], w2_ref[...], _DN,
                                preferred_element_type=jnp.float32)
        h2_ref[:, pl.ds(j * _BK, _BK)] = jnp.maximum(
            h + b2_ref[:, pl.ds(j * _BK, _BK)], 0.0)

    @pl.when(s == 2 * k - 1)
    def _phase3():
        o = jax.lax.dot_general(h2_ref[...], w3_ref[...], _DN,
                                preferred_element_type=jnp.float32)
        o_ref[...] = o + b3_ref[...]


def kernel(x, W1, b1, W2, b2, W3, b3, t):
    del t
    B, D_IN = x.shape
    D_H = W1.shape[0]
    D_OUT = W3.shape[0]
    k = D_H // _BK
    grid = (2 * k,)
    return pl.pallas_call(
        _mlp_kernel,
        grid=grid,
        in_specs=[
            pl.BlockSpec((B, D_IN), lambda s: (0, 0)),
            pl.BlockSpec((_BK, D_IN), lambda s: (jnp.minimum(s, k - 1), 0)),
            pl.BlockSpec((1, D_H), lambda s: (0, 0)),
            pl.BlockSpec((_BK, D_H), lambda s: (jnp.maximum(s - k, 0), 0)),
            pl.BlockSpec((1, D_H), lambda s: (0, 0)),
            pl.BlockSpec((D_OUT, D_H), lambda s: (0, 0)),
            pl.BlockSpec((1, D_OUT), lambda s: (0, 0)),
        ],
        out_specs=pl.BlockSpec((B, D_OUT), lambda s: (0, 0)),
        out_shape=jax.ShapeDtypeStruct((B, D_OUT), jnp.float32),
        scratch_shapes=[
            pltpu.VMEM((B, D_H), jnp.float32),
            pltpu.VMEM((B, D_H), jnp.float32),
        ],
        compiler_params=pltpu.CompilerParams(
            dimension_semantics=("arbitrary",),
        ),
    )(x, W1, b1.reshape(1, -1), W2, b2.reshape(1, -1), W3, b3.reshape(1, -1))


# speedup vs baseline: 1.1200x; 1.1200x over previous
import jax
import jax.numpy as jnp
from jax.experimental import pallas as pl
from jax.experimental.pallas import tpu as pltpu

_BK = 256
_DN = (((1,), (1,)), ((), ()))


def _mlp_kernel(x_ref, w1_ref, b1_ref, w2_ref, b2_ref, w3_ref, b3_ref,
                o_ref, acc_ref):
    s = pl.program_id(0)
    k = pl.num_programs(0)

    h1 = jax.lax.dot_general(x_ref[...], w1_ref[...], _DN,
                             preferred_element_type=jnp.float32)
    h1 = jnp.maximum(h1 + b1_ref[...], 0.0)

    part = jax.lax.dot_general(h1, w2_ref[...], _DN,
                               preferred_element_type=jnp.float32)

    @pl.when(s == 0)
    def _init():
        acc_ref[...] = part + b2_ref[...]

    @pl.when(s > 0)
    def _accum():
        acc_ref[...] += part

    @pl.when(s == k - 1)
    def _final():
        h2 = jnp.maximum(acc_ref[...], 0.0)
        o = jax.lax.dot_general(h2, w3_ref[...], _DN,
                                preferred_element_type=jnp.float32)
        o_ref[...] = o + b3_ref[...]


def kernel(x, W1, b1, W2, b2, W3, b3, t):
    del t
    B, D_IN = x.shape
    D_H = W1.shape[0]
    D_OUT = W3.shape[0]
    k = D_H // _BK
    return pl.pallas_call(
        _mlp_kernel,
        grid=(k,),
        in_specs=[
            pl.BlockSpec((B, D_IN), lambda s: (0, 0)),
            pl.BlockSpec((_BK, D_IN), lambda s: (s, 0)),
            pl.BlockSpec((1, _BK), lambda s: (0, s)),
            pl.BlockSpec((D_H, _BK), lambda s: (0, s)),
            pl.BlockSpec((1, D_H), lambda s: (0, 0)),
            pl.BlockSpec((D_OUT, D_H), lambda s: (0, 0)),
            pl.BlockSpec((1, D_OUT), lambda s: (0, 0)),
        ],
        out_specs=pl.BlockSpec((B, D_OUT), lambda s: (0, 0)),
        out_shape=jax.ShapeDtypeStruct((B, D_OUT), jnp.float32),
        scratch_shapes=[pltpu.VMEM((B, D_H), jnp.float32)],
        compiler_params=pltpu.CompilerParams(
            dimension_semantics=("arbitrary",),
        ),
    )(x, W1, b1.reshape(1, -1), W2, b2.reshape(1, -1), W3, b3.reshape(1, -1))
